# baseline (speedup 1.0000x reference)
import functools

import jax
import jax.numpy as jnp
from jax import lax
from jax.experimental import pallas as pl
from jax.experimental.pallas import tpu as pltpu
from jax.experimental.pallas import tpu_sc as plsc

_NUM_CORES = 2
_NUM_SUBCORES = 16
_NUM_WORKERS = _NUM_CORES * _NUM_SUBCORES
_BROW = 128


def _sc_gather_tiled(idx_flat, wte, n_h, n_bt):
    n = idx_flat.shape[0]
    d = wte.shape[1]
    d_t = d // 8
    per_worker_groups = (n_h * n_bt) // _NUM_WORKERS
    per_worker_idx = per_worker_groups * _BROW

    mesh = plsc.VectorSubcoreMesh(
        core_axis_name="c",
        subcore_axis_name="s",
        num_cores=_NUM_CORES,
        num_subcores=_NUM_SUBCORES,
    )

    @functools.partial(
        pl.kernel,
        out_type=jax.ShapeDtypeStruct((n_h, d_t, n_bt, 8, _BROW), jnp.float32),
        mesh=mesh,
        scratch_types=[
            pltpu.VMEM((per_worker_idx,), jnp.int32),
            pltpu.VMEM((2, _BROW, d), jnp.float32),
            pltpu.VMEM((2, d_t, 8, _BROW), jnp.float32),
            [pltpu.SemaphoreType.DMA] * 2,
            [pltpu.SemaphoreType.DMA] * 2,
        ],
        compiler_params=pltpu.CompilerParams(
            use_tc_tiling_on_sc=False, needs_layout_passes=False
        ),
    )
    def gather_kernel(idx_hbm, table_hbm, out_hbm, idx_v, rows_v, tiles_v,
                      gsems, wsems):
        wid = lax.axis_index("s") * _NUM_CORES + lax.axis_index("c")
        gbase = wid * per_worker_groups

        pltpu.sync_copy(idx_hbm.at[pl.ds(gbase * _BROW, per_worker_idx)], idx_v)

        def fire_gather(g, q):
            pltpu.async_copy(
                table_hbm.at[idx_v.at[pl.ds(g * _BROW, _BROW)]],
                rows_v.at[q],
                gsems[q],
            )

        def wait_gather(q):
            pltpu.make_async_copy(
                table_hbm.at[idx_v.at[pl.ds(0, _BROW)]],
                rows_v.at[q],
                gsems[q],
            ).wait()

        def fire_writes(g, q):
            gg = gbase + g
            h = gg // n_bt
            bt = gg % n_bt
            for jt in range(d_t):
                pltpu.async_copy(
                    tiles_v.at[q, jt],
                    out_hbm.at[h, jt, bt],
                    wsems[q],
                )

        def wait_writes(q):
            for jt in range(d_t):
                pltpu.make_async_copy(
                    tiles_v.at[q, jt],
                    out_hbm.at[0, jt, 0],
                    wsems[q],
                ).wait()

        def transpose_group(q):
            iota = lax.iota(jnp.int32, 16)
            for jt in range(d_t):
                for jr in range(8):
                    col = jnp.full((16,), 8 * jt + jr, jnp.int32)
                    for b0 in range(_BROW // 16):
                        vals = plsc.load_gather(
                            rows_v.at[q], [b0 * 16 + iota, col]
                        )
                        tiles_v[q, jt, jr, pl.ds(b0 * 16, 16)] = vals

        fire_gather(0, 0)

        def pair_body(p, carry):
            for parity in range(2):
                g = 2 * p + parity

                @pl.when(g + 1 < per_worker_groups)
                def _():
                    fire_gather(g + 1, 1 - parity)

                wait_gather(parity)

                @pl.when(g >= 2)
                def _():
                    wait_writes(parity)

                transpose_group(parity)
                fire_writes(g, parity)
            return carry

        lax.fori_loop(0, per_worker_groups // 2, pair_body, 0)
        wait_writes(0)
        wait_writes(1)

    return gather_kernel(idx_flat, wte)


@jax.jit
def _embed(x, wte):
    b, h = x.shape
    v, d = wte.shape
    n_bt = b // _BROW
    idx_flat = x.T.reshape(h * b).astype(jnp.int32)
    out5d = _sc_gather_tiled(idx_flat, wte, h, n_bt)
    return out5d.transpose((2, 4, 0, 1, 3)).reshape(b, h, d)


def kernel(x, wte):
    return _embed(x, wte)


# speedup vs baseline: 2.5614x; 2.5614x over previous
import functools

import jax
import jax.numpy as jnp
from jax import lax
from jax.experimental import pallas as pl
from jax.experimental.pallas import tpu as pltpu
from jax.experimental.pallas import tpu_sc as plsc

_NUM_CORES = 2
_NUM_SUBCORES = 16
_NUM_WORKERS = _NUM_CORES * _NUM_SUBCORES
_BROW = 128


def _sc_gather_tiled(idx_flat, wte, n_h, n_bt):
    n = idx_flat.shape[0]
    d = wte.shape[1]
    d_t = d // 8
    per_worker_groups = (n_h * n_bt) // _NUM_WORKERS
    per_worker_idx = per_worker_groups * _BROW

    mesh = plsc.VectorSubcoreMesh(
        core_axis_name="c",
        subcore_axis_name="s",
        num_cores=_NUM_CORES,
        num_subcores=_NUM_SUBCORES,
    )

    @functools.partial(
        pl.kernel,
        out_type=jax.ShapeDtypeStruct((n_h, d_t, n_bt, 8, _BROW), jnp.float32),
        mesh=mesh,
        scratch_types=[
            pltpu.VMEM((per_worker_idx,), jnp.int32),
            pltpu.VMEM((2, _BROW, d), jnp.float32),
            pltpu.VMEM((2, d_t, 8, _BROW + 1), jnp.float32),
            [pltpu.SemaphoreType.DMA] * 2,
            [pltpu.SemaphoreType.DMA] * 2,
        ],
        compiler_params=pltpu.CompilerParams(
            use_tc_tiling_on_sc=False, needs_layout_passes=False
        ),
    )
    def gather_kernel(idx_hbm, table_hbm, out_hbm, idx_v, rows_v, tiles_v,
                      gsems, wsems):
        wid = lax.axis_index("s") * _NUM_CORES + lax.axis_index("c")
        gbase = wid * per_worker_groups

        pltpu.sync_copy(idx_hbm.at[pl.ds(gbase * _BROW, per_worker_idx)], idx_v)

        def fire_gather(g, q):
            pltpu.async_copy(
                table_hbm.at[idx_v.at[pl.ds(g * _BROW, _BROW)]],
                rows_v.at[q],
                gsems[q],
            )

        def wait_gather(q):
            pltpu.make_async_copy(
                table_hbm.at[idx_v.at[pl.ds(0, _BROW)]],
                rows_v.at[q],
                gsems[q],
            ).wait()

        def fire_writes(g, q):
            gg = gbase + g
            h = gg // n_bt
            bt = gg % n_bt
            for jt in range(d_t):
                pltpu.async_copy(
                    tiles_v.at[q, jt, :, pl.ds(0, _BROW)],
                    out_hbm.at[h, jt, bt],
                    wsems[q],
                )

        def wait_writes(q):
            for jt in range(d_t):
                pltpu.make_async_copy(
                    tiles_v.at[q, jt, :, pl.ds(0, _BROW)],
                    out_hbm.at[0, jt, 0],
                    wsems[q],
                ).wait()

        def transpose_group(q):
            iota = lax.iota(jnp.int32, 16)
            jt_vecs = [(16 * j0 + iota) // 8 for j0 in range(d // 16)]
            jr_vecs = [(16 * j0 + iota) % 8 for j0 in range(d // 16)]
            tiles = tiles_v.at[q]

            def body(b):
                bb = jnp.full((16,), b, jnp.int32)
                for j0 in range(d // 16):
                    vals = rows_v[q, b, pl.ds(16 * j0, 16)]
                    plsc.store_scatter(
                        tiles, [jt_vecs[j0], jr_vecs[j0], bb], vals
                    )

            plsc.parallel_loop(0, _BROW, 1, unroll=8)(body)

        fire_gather(0, 0)

        def pair_body(p, carry):
            for parity in range(2):
                g = 2 * p + parity

                @pl.when(g + 1 < per_worker_groups)
                def _():
                    fire_gather(g + 1, 1 - parity)

                wait_gather(parity)

                @pl.when(g >= 2)
                def _():
                    wait_writes(parity)

                transpose_group(parity)
                fire_writes(g, parity)
            return carry

        lax.fori_loop(0, per_worker_groups // 2, pair_body, 0)
        wait_writes(0)
        wait_writes(1)

    return gather_kernel(idx_flat, wte)


@jax.jit
def _embed(x, wte):
    b, h = x.shape
    v, d = wte.shape
    n_bt = b // _BROW
    idx_flat = x.T.reshape(h * b).astype(jnp.int32)
    out5d = _sc_gather_tiled(idx_flat, wte, h, n_bt)
    return out5d.transpose((2, 4, 0, 1, 3)).reshape(b, h, d)


def kernel(x, wte):
    return _embed(x, wte)


# speedup vs baseline: 4.3135x; 1.6840x over previous
import functools

import jax
import jax.numpy as jnp
from jax import lax
from jax.experimental import pallas as pl
from jax.experimental.pallas import tpu as pltpu
from jax.experimental.pallas import tpu_sc as plsc

_NUM_CORES = 2
_NUM_SUBCORES = 16
_NUM_WORKERS = _NUM_CORES * _NUM_SUBCORES
_BROW = 128


def _sc_untile_transpose(wte_t, tail_flat):
    d, v = wte_t.shape
    n_full = v // 128
    tail = v - n_full * 128
    per_worker = n_full // _NUM_WORKERS + 1
    steps = per_worker + per_worker % 2

    mesh = plsc.VectorSubcoreMesh(
        core_axis_name="c",
        subcore_axis_name="s",
        num_cores=_NUM_CORES,
        num_subcores=_NUM_SUBCORES,
    )

    @functools.partial(
        pl.kernel,
        out_type=jax.ShapeDtypeStruct((v * d,), jnp.float32),
        mesh=mesh,
        scratch_types=[
            [pltpu.VMEM((d, 128), jnp.float32)] * 2,
            [pltpu.VMEM((128 * d,), jnp.float32)] * 2,
            [pltpu.SemaphoreType.DMA] * 2,
            [pltpu.SemaphoreType.DMA] * 2,
        ],
        compiler_params=pltpu.CompilerParams(
            use_tc_tiling_on_sc=True, needs_layout_passes=False
        ),
    )
    def untile_kernel(src_hbm, tail_hbm, out_hbm, chunk_v, rows_v, gsems, wsems):
        wid = lax.axis_index("s") * _NUM_CORES + lax.axis_index("c")

        iota = lax.iota(jnp.int32, 16)
        rvecs = [(iota + r) % 16 for r in range(16)]
        svecs = [64 * iota + ((iota + r) % 16) for r in range(16)]

        def chunk_of(k):
            return wid + _NUM_WORKERS * k

        def fire_in(k, q, width):
            c = chunk_of(k)
            pltpu.async_copy(
                src_hbm.at[:, pl.ds(c * 128, width)],
                chunk_v[q].at[:, pl.ds(0, width)],
                gsems[q],
            )

        def wait_in(q, width):
            pltpu.make_async_copy(
                src_hbm.at[:, pl.ds(0, width)],
                chunk_v[q].at[:, pl.ds(0, width)],
                gsems[q],
            ).wait()

        def fire_out(k, q, width):
            c = chunk_of(k)
            pltpu.async_copy(
                rows_v[q].at[pl.ds(0, width * d)],
                out_hbm.at[pl.ds(c * 128 * d, width * d)],
                wsems[q],
            )

        def wait_out(q, width):
            pltpu.make_async_copy(
                rows_v[q].at[pl.ds(0, width * d)],
                out_hbm.at[pl.ds(0, width * d)],
                wsems[q],
            ).wait()

        def transpose_chunk(q, n_b0):
            rows = rows_v[q]
            chunk = chunk_v[q]

            def body(t):
                j1 = t // n_b0
                b0 = t % n_b0
                col = jnp.full((16,), 16 * b0, jnp.int32) + iota
                sbase = jnp.full((16,), 16 * b0 * d + 16 * j1, jnp.int32)
                rbase = jnp.full((16,), 16 * j1, jnp.int32)
                for r in range(16):
                    vals = plsc.load_gather(chunk, [rbase + rvecs[r], col])
                    plsc.store_scatter(rows, [sbase + svecs[r]], vals)

            plsc.parallel_loop(0, (d // 16) * n_b0, 1, unroll=2)(body)

        def valid(k):
            return chunk_of(k) < n_full

        fire_in(0, 0, 128)

        def pair_body(p, carry):
            for parity in range(2):
                k = 2 * p + parity

                @pl.when(valid(k + 1))
                def _():
                    fire_in(k + 1, 1 - parity, 128)

                @pl.when(valid(k))
                def _():
                    wait_in(parity, 128)

                    @pl.when(k >= 2)
                    def _():
                        wait_out(parity, 128)

                    transpose_chunk(parity, 8)
                    fire_out(k, parity, 128)

            return carry

        lax.fori_loop(0, steps // 2, pair_body, 0)

        wait_out(0, 128)
        wait_out(1, 128)

        if tail:
            @pl.when(wid == 0)
            def _():
                pltpu.sync_copy(tail_hbm, rows_v[0].at[pl.ds(0, tail * d)])
                pltpu.sync_copy(
                    rows_v[0].at[pl.ds(0, tail * d)],
                    out_hbm.at[pl.ds(n_full * 128 * d, tail * d)],
                )

    return untile_kernel(wte_t, tail_flat)


def _sc_gather_tiled(idx_flat, wte, n_h, n_bt):
    n = idx_flat.shape[0]
    d = wte.shape[1]
    d_t = d // 8
    per_worker_groups = (n_h * n_bt) // _NUM_WORKERS
    per_worker_idx = per_worker_groups * _BROW

    mesh = plsc.VectorSubcoreMesh(
        core_axis_name="c",
        subcore_axis_name="s",
        num_cores=_NUM_CORES,
        num_subcores=_NUM_SUBCORES,
    )

    @functools.partial(
        pl.kernel,
        out_type=jax.ShapeDtypeStruct((n_h, d_t, n_bt, 8, _BROW), jnp.float32),
        mesh=mesh,
        scratch_types=[
            pltpu.VMEM((per_worker_idx,), jnp.int32),
            pltpu.VMEM((2, _BROW, d), jnp.float32),
            pltpu.VMEM((2, d_t, 8, _BROW + 1), jnp.float32),
            [pltpu.SemaphoreType.DMA] * 2,
            [pltpu.SemaphoreType.DMA] * 2,
        ],
        compiler_params=pltpu.CompilerParams(
            use_tc_tiling_on_sc=False, needs_layout_passes=False
        ),
    )
    def gather_kernel(idx_hbm, table_hbm, out_hbm, idx_v, rows_v, tiles_v,
                      gsems, wsems):
        wid = lax.axis_index("s") * _NUM_CORES + lax.axis_index("c")
        gbase = wid * per_worker_groups

        pltpu.sync_copy(idx_hbm.at[pl.ds(gbase * _BROW, per_worker_idx)], idx_v)

        def fire_gather(g, q):
            pltpu.async_copy(
                table_hbm.at[idx_v.at[pl.ds(g * _BROW, _BROW)]],
                rows_v.at[q],
                gsems[q],
            )

        def wait_gather(q):
            pltpu.make_async_copy(
                table_hbm.at[idx_v.at[pl.ds(0, _BROW)]],
                rows_v.at[q],
                gsems[q],
            ).wait()

        def fire_writes(g, q):
            gg = gbase + g
            h = gg // n_bt
            bt = gg % n_bt
            for jt in range(d_t):
                pltpu.async_copy(
                    tiles_v.at[q, jt, :, pl.ds(0, _BROW)],
                    out_hbm.at[h, jt, bt],
                    wsems[q],
                )

        def wait_writes(q):
            for jt in range(d_t):
                pltpu.make_async_copy(
                    tiles_v.at[q, jt, :, pl.ds(0, _BROW)],
                    out_hbm.at[0, jt, 0],
                    wsems[q],
                ).wait()

        def transpose_group(q):
            iota = lax.iota(jnp.int32, 16)
            jt_vecs = [(16 * j0 + iota) // 8 for j0 in range(d // 16)]
            jr_vecs = [(16 * j0 + iota) % 8 for j0 in range(d // 16)]
            tiles = tiles_v.at[q]

            def body(b):
                bb = jnp.full((16,), b, jnp.int32)
                for j0 in range(d // 16):
                    vals = rows_v[q, b, pl.ds(16 * j0, 16)]
                    plsc.store_scatter(
                        tiles, [jt_vecs[j0], jr_vecs[j0], bb], vals
                    )

            plsc.parallel_loop(0, _BROW, 1, unroll=8)(body)

        fire_gather(0, 0)

        def pair_body(p, carry):
            for parity in range(2):
                g = 2 * p + parity

                @pl.when(g + 1 < per_worker_groups)
                def _():
                    fire_gather(g + 1, 1 - parity)

                wait_gather(parity)

                @pl.when(g >= 2)
                def _():
                    wait_writes(parity)

                transpose_group(parity)
                fire_writes(g, parity)
            return carry

        lax.fori_loop(0, per_worker_groups // 2, pair_body, 0)
        wait_writes(0)
        wait_writes(1)

    return gather_kernel(idx_flat, wte)


@jax.jit
def _embed(x, wte):
    b, h = x.shape
    v, d = wte.shape
    n_bt = b // _BROW
    idx_flat = x.T.reshape(h * b).astype(jnp.int32)
    n_tail = v % 128
    tail_flat = wte[v - n_tail:, :].reshape(n_tail * d)
    table_flat = _sc_untile_transpose(wte.T, tail_flat)
    table = table_flat.reshape(v, d)
    out5d = _sc_gather_tiled(idx_flat, table, h, n_bt)
    return out5d.transpose((2, 4, 0, 1, 3)).reshape(b, h, d)


def kernel(x, wte):
    return _embed(x, wte)
